# baseline (speedup 1.0000x reference)
import functools

import jax
import jax.numpy as jnp
from jax import lax
from jax.experimental import pallas as pl
from jax.experimental.pallas import tpu as pltpu
from jax.experimental.pallas import tpu_sc as plsc

F32 = jnp.float32
I32 = jnp.int32

NN = 10000
NE = 160000
NP = 10240
TRASH = 10200
EPT = 180224
K = 32
NB1 = 352
NB2 = 176
KD = 64
NBD = 88

_MESH = plsc.VectorSubcoreMesh(core_axis_name="c", subcore_axis_name="s")


@functools.partial(
    pl.kernel,
    out_type=(jax.ShapeDtypeStruct((NP,), F32),
              jax.ShapeDtypeStruct((NP,), F32)),
    mesh=_MESH,
    scratch_types=[
        pltpu.VMEM_SHARED((NP,), F32),
        pltpu.VMEM((NBD * KD,), I32),
        pltpu.VMEM((KD,), F32),
        pltpu.VMEM((640,), F32),
    ],
)
def _deg_kernel(dst_e, deg0, deg1, acc, didx, ones, zbuf):
    c = lax.axis_index("c")
    s = lax.axis_index("s")
    w = c * 16 + s

    @pl.loop(0, KD // 16)
    def _(i):
        ones[pl.ds(i * 16, 16)] = jnp.ones((16,), F32)

    @pl.loop(0, 40)
    def _(i):
        zbuf[pl.ds(i * 16, 16)] = jnp.zeros((16,), F32)

    pltpu.sync_copy(zbuf, acc.at[pl.ds(s * 640, 640)])
    pltpu.sync_copy(dst_e.at[pl.ds(w * (NBD * KD), NBD * KD)], didx)
    plsc.subcore_barrier()

    @pl.loop(0, NBD)
    def _(b):
        pltpu.sync_copy(ones, acc.at[didx.at[pl.ds(b * KD, KD)]], add=True)

    plsc.subcore_barrier()

    @pl.when(c == 0)
    def _():
        pltpu.sync_copy(acc.at[pl.ds(s * 640, 640)], deg0.at[pl.ds(s * 640, 640)])

    @pl.when(c == 1)
    def _():
        pltpu.sync_copy(acc.at[pl.ds(s * 640, 640)], deg1.at[pl.ds(s * 640, 640)])


def _gemm1_body(deg_ref, x_ref, w_ref, f1b_ref, rh1_ref, dinv_ref):
    dinv = lax.rsqrt(jnp.maximum(deg_ref[...], 1.0))
    acc = jnp.dot(x_ref[...], w_ref[...], preferred_element_type=F32)
    rh1_ref[0] = acc[:, :1024] + f1b_ref[...]
    rh1_ref[1] = acc[:, 1024:] * dinv
    dinv_ref[...] = dinv


def _gemm1(deg_col, x, wf1, f1b):
    bm = 400
    return pl.pallas_call(
        _gemm1_body,
        grid=(NN // bm,),
        in_specs=[
            pl.BlockSpec((bm, 1), lambda i: (i, 0)),
            pl.BlockSpec((bm, 2813), lambda i: (i, 0)),
            pl.BlockSpec((2813, 2048), lambda i: (0, 0)),
            pl.BlockSpec((1, 1024), lambda i: (0, 0)),
        ],
        out_specs=[
            pl.BlockSpec((2, bm, 1024), lambda i: (0, i, 0)),
            pl.BlockSpec((bm, 1), lambda i: (i, 0)),
        ],
        out_shape=[
            jax.ShapeDtypeStruct((2, NN, 1024), F32),
            jax.ShapeDtypeStruct((NN, 1), F32),
        ],
    )(deg_col, x, wf1, f1b)


def _edge_sweep(nb, tbl, acc, sidx, didx, rows, gsem, ssem):
    def gth(b, sl):
        return pltpu.async_copy(tbl.at[sidx.at[pl.ds(b * K, K)]],
                                rows.at[sl], gsem.at[sl])

    def gth_make(b, sl):
        return pltpu.make_async_copy(tbl.at[sidx.at[pl.ds(b * K, K)]],
                                     rows.at[sl], gsem.at[sl])

    def sct_make(b, sl):
        return pltpu.make_async_copy(rows.at[sl],
                                     acc.at[didx.at[pl.ds(b * K, K)]],
                                     ssem.at[sl])

    for b in range(4):
        gth(b, b)

    @pl.loop(0, nb)
    def _(b):
        sl = lax.rem(b, 6)
        gth_make(b, sl).wait()
        pltpu.async_copy(rows.at[sl], acc.at[didx.at[pl.ds(b * K, K)]],
                         ssem.at[sl], add=True)

        @pl.when(b + 4 < nb)
        def _():
            sl2 = lax.rem(b + 4, 6)

            @pl.when(b >= 2)
            def _():
                sct_make(b - 2, sl2).wait()

            gth(b + 4, sl2)

    for t in range(6):
        b = nb - 6 + t
        sct_make(b, b % 6).wait()


def _zero_acc(acc, zbuf, s, zsem):
    @pl.loop(0, 80)
    def _(z):
        pltpu.async_copy(zbuf, acc.at[pl.ds(s * 640 + z * 8, 8)], zsem)

    @pl.loop(0, 80)
    def _(z):
        pltpu.make_async_copy(zbuf, acc.at[pl.ds(s * 640 + z * 8, 8)],
                              zsem).wait()


def _fill_zbuf(zbuf):
    @pl.loop(0, 8)
    def _(i):
        @pl.loop(0, 8)
        def _(j):
            zbuf[i, pl.ds(j * 16, 16)] = jnp.zeros((16,), F32)


@functools.partial(
    pl.kernel,
    out_type=jax.ShapeDtypeStruct((NP, 1024), F32),
    mesh=_MESH,
    scratch_types=[
        pltpu.VMEM_SHARED((NP, 128), F32),
        pltpu.VMEM((NB1 * K,), I32),
        pltpu.VMEM((NB1 * K,), I32),
        pltpu.VMEM((6, K, 128), F32),
        pltpu.VMEM((8, 128), F32),
        pltpu.SemaphoreType.DMA((6,)),
        pltpu.SemaphoreType.DMA((6,)),
        pltpu.SemaphoreType.DMA,
    ],
)
def _prop1_kernel(rh1, src_e, dst_e, out1, acc, sidx, didx, rows, zbuf,
                  gsem, ssem, zsem):
    c = lax.axis_index("c")
    s = lax.axis_index("s")
    accv = acc
    _fill_zbuf(zbuf)
    pltpu.sync_copy(src_e.at[pl.ds(s * (NB1 * K), NB1 * K)], sidx)
    pltpu.sync_copy(dst_e.at[pl.ds(s * (NB1 * K), NB1 * K)], didx)

    for k in range(4):
        col0 = (c * 4 + k) * 128
        _zero_acc(accv, zbuf, s, zsem)
        plsc.subcore_barrier()
        tbl = rh1.at[1, :, pl.ds(col0, 128)]
        _edge_sweep(NB1, tbl, accv, sidx, didx, rows, gsem, ssem)
        plsc.subcore_barrier()
        pltpu.sync_copy(accv.at[pl.ds(s * 640, 640)],
                        out1.at[pl.ds(s * 640, 640), pl.ds(col0, 128)])
        plsc.subcore_barrier()


@functools.partial(
    pl.kernel,
    out_type=jax.ShapeDtypeStruct((2, NP, 512), F32),
    mesh=_MESH,
    scratch_types=[
        pltpu.VMEM_SHARED((NP, 128), F32),
        pltpu.VMEM((NB2 * K,), I32),
        pltpu.VMEM((NB2 * K,), I32),
        pltpu.VMEM((6, K, 128), F32),
        pltpu.VMEM((8, 128), F32),
        pltpu.SemaphoreType.DMA((6,)),
        pltpu.SemaphoreType.DMA((6,)),
        pltpu.SemaphoreType.DMA,
    ],
)
def _prop2_kernel(h2p, src_e, dst_e, out2, acc, sidx, didx, rows, zbuf,
                  gsem, ssem, zsem):
    c = lax.axis_index("c")
    s = lax.axis_index("s")
    w = c * 16 + s
    accv = acc
    _fill_zbuf(zbuf)
    pltpu.sync_copy(src_e.at[pl.ds(w * (NB2 * K), NB2 * K)], sidx)
    pltpu.sync_copy(dst_e.at[pl.ds(w * (NB2 * K), NB2 * K)], didx)

    for k in range(4):
        col0 = k * 128
        _zero_acc(accv, zbuf, s, zsem)
        plsc.subcore_barrier()
        tbl = h2p.at[:, pl.ds(col0, 128)]
        _edge_sweep(NB2, tbl, accv, sidx, didx, rows, gsem, ssem)
        plsc.subcore_barrier()
        pltpu.sync_copy(accv.at[pl.ds(s * 640, 640)],
                        out2.at[c, pl.ds(s * 640, 640), pl.ds(col0, 128)])
        plsc.subcore_barrier()


def _gemm2_body(rh1_ref, out1_ref, dinv_ref, b1_ref, w_ref, f2b_ref,
                r2_ref, h2p_ref):
    h = jnp.maximum(
        rh1_ref[0] + dinv_ref[...] * out1_ref[...] + b1_ref[...], 0.0)
    acc = jnp.dot(h, w_ref[...], preferred_element_type=F32)
    r2_ref[...] = acc[:, :400] + f2b_ref[...]
    h2p_ref[...] = jnp.concatenate(
        [acc[:, 400:] * dinv_ref[...], jnp.zeros((h.shape[0], 112), F32)],
        axis=1)


def _gemm2(rh1, out1, dinv, b1, wf2, f2b):
    bm = 1000
    return pl.pallas_call(
        _gemm2_body,
        grid=(NN // bm,),
        in_specs=[
            pl.BlockSpec((1, bm, 1024), lambda i: (0, i, 0)),
            pl.BlockSpec((bm, 1024), lambda i: (i, 0)),
            pl.BlockSpec((bm, 1), lambda i: (i, 0)),
            pl.BlockSpec((1, 1024), lambda i: (0, 0)),
            pl.BlockSpec((1024, 800), lambda i: (0, 0)),
            pl.BlockSpec((1, 400), lambda i: (0, 0)),
        ],
        out_specs=[
            pl.BlockSpec((bm, 400), lambda i: (i, 0)),
            pl.BlockSpec((bm, 512), lambda i: (i, 0)),
        ],
        out_shape=[
            jax.ShapeDtypeStruct((NN, 400), F32),
            jax.ShapeDtypeStruct((NN, 512), F32),
        ],
    )(rh1, out1, dinv, b1, wf2, f2b)


def _comb_body(r2_ref, out2_ref, dinv_ref, b2_ref, hfin_ref):
    o = out2_ref[0, :, :400] + out2_ref[1, :, :400]
    h = jnp.maximum(r2_ref[...] + dinv_ref[...] * o + b2_ref[...], 0.0)
    hfin_ref[...] = jnp.concatenate(
        [h, jnp.zeros((h.shape[0], 112), F32)], axis=1)


def _comb(r2, out2, dinv, b2):
    bm = 2000
    return pl.pallas_call(
        _comb_body,
        grid=(NN // bm,),
        in_specs=[
            pl.BlockSpec((bm, 400), lambda i: (i, 0)),
            pl.BlockSpec((2, bm, 512), lambda i: (0, i, 0)),
            pl.BlockSpec((bm, 1), lambda i: (i, 0)),
            pl.BlockSpec((1, 400), lambda i: (0, 0)),
        ],
        out_specs=pl.BlockSpec((bm, 512), lambda i: (i, 0)),
        out_shape=jax.ShapeDtypeStruct((NN, 512), F32),
    )(r2, out2, dinv, b2)


@functools.partial(
    pl.kernel,
    out_type=jax.ShapeDtypeStruct((4096, 1024), F32),
    mesh=_MESH,
    scratch_types=[
        pltpu.VMEM((128,), I32),
        pltpu.VMEM((128,), I32),
        pltpu.VMEM((128, 512), F32),
        pltpu.SemaphoreType.DMA((1,)),
    ],
)
def _pairs_kernel(hfin, lidx, ridx, feat, li, ri, buf, sems):
    c = lax.axis_index("c")
    s = lax.axis_index("s")
    w = c * 16 + s
    p0 = w * 128
    pltpu.sync_copy(lidx.at[pl.ds(p0, 128)], li)
    pltpu.sync_copy(ridx.at[pl.ds(p0, 128)], ri)
    pltpu.async_copy(hfin.at[li], buf, sems.at[0])
    pltpu.make_async_copy(hfin.at[li], buf, sems.at[0]).wait()
    pltpu.sync_copy(buf, feat.at[pl.ds(p0, 128), pl.ds(0, 512)])
    pltpu.async_copy(hfin.at[ri], buf, sems.at[0])
    pltpu.make_async_copy(hfin.at[ri], buf, sems.at[0]).wait()
    pltpu.sync_copy(buf, feat.at[pl.ds(p0, 128), pl.ds(512, 512)])


def _mlp_body(feat_ref, w1_ref, b1_ref, w2_ref, b2_ref, w3_ref, b3_ref, z_ref):
    z1 = (jnp.dot(feat_ref[:, :400], w1_ref[:400], preferred_element_type=F32)
          + jnp.dot(feat_ref[:, 512:912], w1_ref[400:],
                    preferred_element_type=F32))
    z1 = jnp.maximum(z1 + b1_ref[...], 0.0)
    z2 = jnp.maximum(
        jnp.dot(z1, w2_ref[...], preferred_element_type=F32) + b2_ref[...], 0.0)
    z3 = jnp.dot(z2, w3_ref[...], preferred_element_type=F32) + b3_ref[...]
    z_ref[...] = jax.nn.sigmoid(z3)


def _mlp(feat, w1, b1, w2, b2, w3, b3):
    return pl.pallas_call(
        _mlp_body,
        out_shape=jax.ShapeDtypeStruct((4096, 1), F32),
    )(feat, w1, b1, w2, b2, w3, b3)


def kernel(x, a, sample_train, conv1_W, conv1_b, conv2_W, conv2_b,
           f1_W, f1_b, f2_W, f2_b, fc1_W, fc1_b, fc2_W, fc2_b, fc3_W, fc3_b):
    loop = jnp.arange(NN, dtype=I32)
    pad = EPT - NE - NN
    src_e = jnp.concatenate([a[0], loop, jnp.zeros((pad,), I32)])
    dst_e = jnp.concatenate([a[1], loop, jnp.full((pad,), TRASH, I32)])

    deg0, deg1 = _deg_kernel(dst_e)
    deg_col = (deg0 + deg1)[:, None]

    wf1 = jnp.concatenate([f1_W, conv1_W], axis=1)
    rh1, dinv = _gemm1(deg_col, x, wf1, f1_b.reshape(1, -1))

    out1 = _prop1_kernel(rh1, src_e, dst_e)

    wf2 = jnp.concatenate([f2_W, conv2_W], axis=1)
    r2, h2p = _gemm2(rh1, out1, dinv, conv1_b.reshape(1, -1), wf2,
                     f2_b.reshape(1, -1))

    out2 = _prop2_kernel(h2p, src_e, dst_e)
    hfin = _comb(r2, out2, dinv, conv2_b.reshape(1, -1))

    lidx = sample_train[:, 0]
    ridx = sample_train[:, 1] + 5000
    feat = _pairs_kernel(hfin, lidx, ridx)

    z = _mlp(feat, fc1_W, fc1_b.reshape(1, -1), fc2_W, fc2_b.reshape(1, -1),
             fc3_W, fc3_b.reshape(1, -1))
    return z


# speedup vs baseline: 1.6002x; 1.6002x over previous
import functools

import jax
import jax.numpy as jnp
from jax import lax
from jax.experimental import pallas as pl
from jax.experimental.pallas import tpu as pltpu
from jax.experimental.pallas import tpu_sc as plsc

F32 = jnp.float32
I32 = jnp.int32

NN = 10000
NE = 160000
NP = 10240
TRASH = 10200
EPT = 180224
K = 32
KA = 16
EPTW = EPT // 32
EPTS = EPT // 16
NBA = EPTW // KA
NBS = EPTS // K
KD = 64
NBD = 88

_MESH = plsc.VectorSubcoreMesh(core_axis_name="c", subcore_axis_name="s")


@functools.partial(
    pl.kernel,
    out_type=(jax.ShapeDtypeStruct((NP,), F32),
              jax.ShapeDtypeStruct((NP,), F32)),
    mesh=_MESH,
    scratch_types=[
        pltpu.VMEM_SHARED((NP,), F32),
        pltpu.VMEM((NBD * KD,), I32),
        pltpu.VMEM((KD,), F32),
        pltpu.VMEM((640,), F32),
    ],
)
def _deg_kernel(dst_e, deg0, deg1, acc, didx, ones, zbuf):
    c = lax.axis_index("c")
    s = lax.axis_index("s")
    w = c * 16 + s

    @pl.loop(0, KD // 16)
    def _(i):
        ones[pl.ds(i * 16, 16)] = jnp.ones((16,), F32)

    @pl.loop(0, 40)
    def _(i):
        zbuf[pl.ds(i * 16, 16)] = jnp.zeros((16,), F32)

    pltpu.sync_copy(zbuf, acc.at[pl.ds(s * 640, 640)])
    pltpu.sync_copy(dst_e.at[pl.ds(w * (NBD * KD), NBD * KD)], didx)
    plsc.subcore_barrier()

    @pl.loop(0, NBD)
    def _(b):
        pltpu.sync_copy(ones, acc.at[didx.at[pl.ds(b * KD, KD)]], add=True)

    plsc.subcore_barrier()

    @pl.when(c == 0)
    def _():
        pltpu.sync_copy(acc.at[pl.ds(s * 640, 640)], deg0.at[pl.ds(s * 640, 640)])

    @pl.when(c == 1)
    def _():
        pltpu.sync_copy(acc.at[pl.ds(s * 640, 640)], deg1.at[pl.ds(s * 640, 640)])


def _gemm1_body(deg_ref, x_ref, w_ref, f1b_ref, rh1_ref, dinv_ref):
    dinv = lax.rsqrt(jnp.maximum(deg_ref[...], 1.0))
    acc = jnp.dot(x_ref[...], w_ref[...], preferred_element_type=F32)
    rh1_ref[0] = acc[:, :1024] + f1b_ref[...]
    rh1_ref[1] = acc[:, 1024:] * dinv
    dinv_ref[...] = dinv


def _gemm1(deg_col, x, wf1, f1b):
    bm = 400
    return pl.pallas_call(
        _gemm1_body,
        grid=(NN // bm,),
        in_specs=[
            pl.BlockSpec((bm, 1), lambda i: (i, 0)),
            pl.BlockSpec((bm, 2813), lambda i: (i, 0)),
            pl.BlockSpec((2813, 2048), lambda i: (0, 0)),
            pl.BlockSpec((1, 1024), lambda i: (0, 0)),
        ],
        out_specs=[
            pl.BlockSpec((2, bm, 1024), lambda i: (0, i, 0)),
            pl.BlockSpec((bm, 1), lambda i: (i, 0)),
        ],
        out_shape=[
            jax.ShapeDtypeStruct((2, NN, 1024), F32),
            jax.ShapeDtypeStruct((NN, 1), F32),
        ],
    )(deg_col, x, wf1, f1b)


def _sweep_pipeline(nb, gth, gth_make, sct_make, sct_start):
    for b in range(4):
        gth(b, b)

    @pl.loop(0, nb)
    def _(b):
        sl = lax.rem(b, 6)
        gth_make(b, sl).wait()
        sct_start(b, sl)

        @pl.when(b + 4 < nb)
        def _():
            sl2 = lax.rem(b + 4, 6)

            @pl.when(b >= 2)
            def _():
                sct_make(b - 2, sl2).wait()

            gth(b + 4, sl2)

    for t in range(6):
        b = nb - 6 + t
        sct_make(b, b % 6).wait()


def _zero_acc(acc, zbuf, s, zsem):
    @pl.loop(0, 80)
    def _(z):
        pltpu.async_copy(zbuf, acc.at[pl.ds(s * 640 + z * 8, 8)], zsem)

    @pl.loop(0, 80)
    def _(z):
        pltpu.make_async_copy(zbuf, acc.at[pl.ds(s * 640 + z * 8, 8)],
                              zsem).wait()


def _fill_zbuf(zbuf):
    @pl.loop(0, 8)
    def _(i):
        @pl.loop(0, 8)
        def _(j):
            zbuf[i, pl.ds(j * 16, 16)] = jnp.zeros((16,), F32)


def _stash_body(tbl, src_e, stash, sidx, rows, gsem, wsem, ncols):
    c = lax.axis_index("c")
    s = lax.axis_index("s")
    w = c * 16 + s
    ebase = w * EPTW
    pltpu.sync_copy(src_e.at[pl.ds(ebase, EPTW)], sidx)

    def gth(b, sl):
        return pltpu.async_copy(tbl.at[sidx.at[pl.ds(b * KA, KA)]],
                                rows.at[sl], gsem.at[sl])

    def gth_make(b, sl):
        return pltpu.make_async_copy(tbl.at[sidx.at[pl.ds(b * KA, KA)]],
                                     rows.at[sl], gsem.at[sl])

    def wrt_make(b, sl):
        return pltpu.make_async_copy(
            rows.at[sl], stash.at[pl.ds(ebase + b * KA, KA), :], wsem.at[sl])

    gth(0, 0)

    @pl.loop(0, NBA)
    def _(b):
        sl = lax.rem(b, 2)
        gth_make(b, sl).wait()

        @pl.when(b + 1 < NBA)
        def _():
            sl2 = lax.rem(b + 1, 2)

            @pl.when(b >= 1)
            def _():
                wrt_make(b - 1, sl2).wait()

            gth(b + 1, sl2)

        pltpu.async_copy(rows.at[sl], stash.at[pl.ds(ebase + b * KA, KA), :],
                         wsem.at[sl])

    for t in range(2):
        b = NBA - 2 + t
        wrt_make(b, b % 2).wait()


@functools.partial(
    pl.kernel,
    out_type=jax.ShapeDtypeStruct((EPT, 1024), F32),
    mesh=_MESH,
    scratch_types=[
        pltpu.VMEM((EPTW,), I32),
        pltpu.VMEM((2, KA, 1024), F32),
        pltpu.SemaphoreType.DMA((2,)),
        pltpu.SemaphoreType.DMA((2,)),
    ],
)
def _stash1_kernel(rh1, src_e, stash, sidx, rows, gsem, wsem):
    _stash_body(rh1.at[1], src_e, stash, sidx, rows, gsem, wsem, 1024)


@functools.partial(
    pl.kernel,
    out_type=jax.ShapeDtypeStruct((EPT, 512), F32),
    mesh=_MESH,
    scratch_types=[
        pltpu.VMEM((EPTW,), I32),
        pltpu.VMEM((2, KA, 512), F32),
        pltpu.SemaphoreType.DMA((2,)),
        pltpu.SemaphoreType.DMA((2,)),
    ],
)
def _stash2_kernel(h2p, src_e, stash, sidx, rows, gsem, wsem):
    _stash_body(h2p, src_e, stash, sidx, rows, gsem, wsem, 512)


def _sweep_body(stash, dst_e, out, acc, didx, rows, zbuf, gsem, ssem, zsem,
                chunks_per_core):
    c = lax.axis_index("c")
    s = lax.axis_index("s")
    _fill_zbuf(zbuf)
    ebase = s * EPTS
    pltpu.sync_copy(dst_e.at[pl.ds(ebase, EPTS)], didx)

    for k in range(chunks_per_core):
        chunk = c * chunks_per_core + k
        col0 = chunk * 128
        _zero_acc(acc, zbuf, s, zsem)
        plsc.subcore_barrier()

        def gth(b, sl):
            return pltpu.async_copy(
                stash.at[pl.ds(ebase + b * K, K), pl.ds(col0, 128)],
                rows.at[sl], gsem.at[sl])

        def gth_make(b, sl):
            return pltpu.make_async_copy(
                stash.at[pl.ds(ebase + b * K, K), pl.ds(col0, 128)],
                rows.at[sl], gsem.at[sl])

        def sct_make(b, sl):
            return pltpu.make_async_copy(rows.at[sl],
                                         acc.at[didx.at[pl.ds(b * K, K)]],
                                         ssem.at[sl])

        def sct_start(b, sl):
            pltpu.async_copy(rows.at[sl], acc.at[didx.at[pl.ds(b * K, K)]],
                             ssem.at[sl], add=True)

        _sweep_pipeline(NBS, gth, gth_make, sct_make, sct_start)
        plsc.subcore_barrier()
        pltpu.sync_copy(acc.at[pl.ds(s * 640, 640)],
                        out.at[pl.ds(s * 640, 640), pl.ds(col0, 128)])
        plsc.subcore_barrier()


@functools.partial(
    pl.kernel,
    out_type=jax.ShapeDtypeStruct((NP, 1024), F32),
    mesh=_MESH,
    scratch_types=[
        pltpu.VMEM_SHARED((NP, 128), F32),
        pltpu.VMEM((EPTS,), I32),
        pltpu.VMEM((6, K, 128), F32),
        pltpu.VMEM((8, 128), F32),
        pltpu.SemaphoreType.DMA((6,)),
        pltpu.SemaphoreType.DMA((6,)),
        pltpu.SemaphoreType.DMA,
    ],
)
def _sweep1_kernel(stash, dst_e, out1, acc, didx, rows, zbuf, gsem, ssem, zsem):
    _sweep_body(stash, dst_e, out1, acc, didx, rows, zbuf, gsem, ssem, zsem, 4)


@functools.partial(
    pl.kernel,
    out_type=jax.ShapeDtypeStruct((NP, 512), F32),
    mesh=_MESH,
    scratch_types=[
        pltpu.VMEM_SHARED((NP, 128), F32),
        pltpu.VMEM((EPTS,), I32),
        pltpu.VMEM((6, K, 128), F32),
        pltpu.VMEM((8, 128), F32),
        pltpu.SemaphoreType.DMA((6,)),
        pltpu.SemaphoreType.DMA((6,)),
        pltpu.SemaphoreType.DMA,
    ],
)
def _sweep2_kernel(stash, dst_e, out2, acc, didx, rows, zbuf, gsem, ssem, zsem):
    _sweep_body(stash, dst_e, out2, acc, didx, rows, zbuf, gsem, ssem, zsem, 2)


def _gemm2_body(rh1_ref, out1_ref, dinv_ref, b1_ref, w_ref, f2b_ref,
                r2_ref, h2p_ref):
    h = jnp.maximum(
        rh1_ref[0] + dinv_ref[...] * out1_ref[...] + b1_ref[...], 0.0)
    acc = jnp.dot(h, w_ref[...], preferred_element_type=F32)
    r2_ref[...] = acc[:, :400] + f2b_ref[...]
    h2p_ref[...] = jnp.concatenate(
        [acc[:, 400:] * dinv_ref[...], jnp.zeros((h.shape[0], 112), F32)],
        axis=1)


def _gemm2(rh1, out1, dinv, b1, wf2, f2b):
    bm = 1000
    return pl.pallas_call(
        _gemm2_body,
        grid=(NN // bm,),
        in_specs=[
            pl.BlockSpec((1, bm, 1024), lambda i: (0, i, 0)),
            pl.BlockSpec((bm, 1024), lambda i: (i, 0)),
            pl.BlockSpec((bm, 1), lambda i: (i, 0)),
            pl.BlockSpec((1, 1024), lambda i: (0, 0)),
            pl.BlockSpec((1024, 800), lambda i: (0, 0)),
            pl.BlockSpec((1, 400), lambda i: (0, 0)),
        ],
        out_specs=[
            pl.BlockSpec((bm, 400), lambda i: (i, 0)),
            pl.BlockSpec((bm, 512), lambda i: (i, 0)),
        ],
        out_shape=[
            jax.ShapeDtypeStruct((NN, 400), F32),
            jax.ShapeDtypeStruct((NN, 512), F32),
        ],
    )(rh1, out1, dinv, b1, wf2, f2b)


def _comb_body(r2_ref, out2_ref, dinv_ref, b2_ref, hfin_ref):
    o = out2_ref[:, :400]
    h = jnp.maximum(r2_ref[...] + dinv_ref[...] * o + b2_ref[...], 0.0)
    hfin_ref[...] = jnp.concatenate(
        [h, jnp.zeros((h.shape[0], 112), F32)], axis=1)


def _comb(r2, out2, dinv, b2):
    bm = 2000
    return pl.pallas_call(
        _comb_body,
        grid=(NN // bm,),
        in_specs=[
            pl.BlockSpec((bm, 400), lambda i: (i, 0)),
            pl.BlockSpec((bm, 512), lambda i: (i, 0)),
            pl.BlockSpec((bm, 1), lambda i: (i, 0)),
            pl.BlockSpec((1, 400), lambda i: (0, 0)),
        ],
        out_specs=pl.BlockSpec((bm, 512), lambda i: (i, 0)),
        out_shape=jax.ShapeDtypeStruct((NN, 512), F32),
    )(r2, out2, dinv, b2)


@functools.partial(
    pl.kernel,
    out_type=jax.ShapeDtypeStruct((4096, 1024), F32),
    mesh=_MESH,
    scratch_types=[
        pltpu.VMEM((128,), I32),
        pltpu.VMEM((128,), I32),
        pltpu.VMEM((128, 512), F32),
        pltpu.SemaphoreType.DMA((1,)),
    ],
)
def _pairs_kernel(hfin, lidx, ridx, feat, li, ri, buf, sems):
    c = lax.axis_index("c")
    s = lax.axis_index("s")
    w = c * 16 + s
    p0 = w * 128
    pltpu.sync_copy(lidx.at[pl.ds(p0, 128)], li)
    pltpu.sync_copy(ridx.at[pl.ds(p0, 128)], ri)
    pltpu.async_copy(hfin.at[li], buf, sems.at[0])
    pltpu.make_async_copy(hfin.at[li], buf, sems.at[0]).wait()
    pltpu.sync_copy(buf, feat.at[pl.ds(p0, 128), pl.ds(0, 512)])
    pltpu.async_copy(hfin.at[ri], buf, sems.at[0])
    pltpu.make_async_copy(hfin.at[ri], buf, sems.at[0]).wait()
    pltpu.sync_copy(buf, feat.at[pl.ds(p0, 128), pl.ds(512, 512)])


def _mlp_body(feat_ref, w1_ref, b1_ref, w2_ref, b2_ref, w3_ref, b3_ref, z_ref):
    z1 = (jnp.dot(feat_ref[:, :400], w1_ref[:400], preferred_element_type=F32)
          + jnp.dot(feat_ref[:, 512:912], w1_ref[400:],
                    preferred_element_type=F32))
    z1 = jnp.maximum(z1 + b1_ref[...], 0.0)
    z2 = jnp.maximum(
        jnp.dot(z1, w2_ref[...], preferred_element_type=F32) + b2_ref[...], 0.0)
    z3 = jnp.dot(z2, w3_ref[...], preferred_element_type=F32) + b3_ref[...]
    z_ref[...] = jax.nn.sigmoid(z3)


def _mlp(feat, w1, b1, w2, b2, w3, b3):
    return pl.pallas_call(
        _mlp_body,
        out_shape=jax.ShapeDtypeStruct((4096, 1), F32),
    )(feat, w1, b1, w2, b2, w3, b3)


def kernel(x, a, sample_train, conv1_W, conv1_b, conv2_W, conv2_b,
           f1_W, f1_b, f2_W, f2_b, fc1_W, fc1_b, fc2_W, fc2_b, fc3_W, fc3_b):
    loop = jnp.arange(NN, dtype=I32)
    pad = EPT - NE - NN
    src_e = jnp.concatenate([a[0], loop, jnp.zeros((pad,), I32)])
    dst_e = jnp.concatenate([a[1], loop, jnp.full((pad,), TRASH, I32)])

    deg0, deg1 = _deg_kernel(dst_e)
    deg_col = (deg0 + deg1)[:, None]

    wf1 = jnp.concatenate([f1_W, conv1_W], axis=1)
    rh1, dinv = _gemm1(deg_col, x, wf1, f1_b.reshape(1, -1))

    stash1 = _stash1_kernel(rh1, src_e)
    out1 = _sweep1_kernel(stash1, dst_e)

    wf2 = jnp.concatenate([f2_W, conv2_W], axis=1)
    r2, h2p = _gemm2(rh1, out1, dinv, conv1_b.reshape(1, -1), wf2,
                     f2_b.reshape(1, -1))

    stash2 = _stash2_kernel(h2p, src_e)
    out2 = _sweep2_kernel(stash2, dst_e)
    hfin = _comb(r2, out2, dinv, conv2_b.reshape(1, -1))

    lidx = sample_train[:, 0]
    ridx = sample_train[:, 1] + 5000
    feat = _pairs_kernel(hfin, lidx, ridx)

    z = _mlp(feat, fc1_W, fc1_b.reshape(1, -1), fc2_W, fc2_b.reshape(1, -1),
             fc3_W, fc3_b.reshape(1, -1))
    return z


# speedup vs baseline: 1.7049x; 1.0654x over previous
import functools

import jax
import jax.numpy as jnp
from jax import lax
from jax.experimental import pallas as pl
from jax.experimental.pallas import tpu as pltpu
from jax.experimental.pallas import tpu_sc as plsc

F32 = jnp.float32
I32 = jnp.int32

NN = 10000
NE = 160000
NP = 10240
TRASH = 10200
EPT = 180224
K = 32
KA = 16
EPTW = EPT // 32
EPTS = EPT // 16
NBA = EPTW // KA
NBS = EPTS // K
KD = 64
NBD = 88

_MESH = plsc.VectorSubcoreMesh(core_axis_name="c", subcore_axis_name="s")


@functools.partial(
    pl.kernel,
    out_type=(jax.ShapeDtypeStruct((NP,), F32),
              jax.ShapeDtypeStruct((NP,), F32)),
    mesh=_MESH,
    scratch_types=[
        pltpu.VMEM_SHARED((NP,), F32),
        pltpu.VMEM((NBD * KD,), I32),
        pltpu.VMEM((KD,), F32),
        pltpu.VMEM((640,), F32),
    ],
)
def _deg_kernel(dst_e, deg0, deg1, acc, didx, ones, zbuf):
    c = lax.axis_index("c")
    s = lax.axis_index("s")
    w = c * 16 + s

    @pl.loop(0, KD // 16)
    def _(i):
        ones[pl.ds(i * 16, 16)] = jnp.ones((16,), F32)

    @pl.loop(0, 40)
    def _(i):
        zbuf[pl.ds(i * 16, 16)] = jnp.zeros((16,), F32)

    pltpu.sync_copy(zbuf, acc.at[pl.ds(s * 640, 640)])
    pltpu.sync_copy(dst_e.at[pl.ds(w * (NBD * KD), NBD * KD)], didx)
    plsc.subcore_barrier()

    @pl.loop(0, NBD)
    def _(b):
        pltpu.sync_copy(ones, acc.at[didx.at[pl.ds(b * KD, KD)]], add=True)

    plsc.subcore_barrier()

    @pl.when(c == 0)
    def _():
        pltpu.sync_copy(acc.at[pl.ds(s * 640, 640)], deg0.at[pl.ds(s * 640, 640)])

    @pl.when(c == 1)
    def _():
        pltpu.sync_copy(acc.at[pl.ds(s * 640, 640)], deg1.at[pl.ds(s * 640, 640)])


def _gemm1_body(deg_ref, x_ref, w_ref, f1b_ref, rh1_ref, dinv_ref):
    dinv = lax.rsqrt(jnp.maximum(deg_ref[...], 1.0))
    acc = jnp.dot(x_ref[...], w_ref[...], preferred_element_type=F32)
    rh1_ref[0] = acc[:, :1024] + f1b_ref[...]
    rh1_ref[1] = acc[:, 1024:] * dinv
    dinv_ref[...] = dinv


def _gemm1(deg_col, x, wf1, f1b):
    bm = 400
    return pl.pallas_call(
        _gemm1_body,
        grid=(NN // bm,),
        in_specs=[
            pl.BlockSpec((bm, 1), lambda i: (i, 0)),
            pl.BlockSpec((bm, 2813), lambda i: (i, 0)),
            pl.BlockSpec((2813, 2048), lambda i: (0, 0)),
            pl.BlockSpec((1, 1024), lambda i: (0, 0)),
        ],
        out_specs=[
            pl.BlockSpec((2, bm, 1024), lambda i: (0, i, 0)),
            pl.BlockSpec((bm, 1), lambda i: (i, 0)),
        ],
        out_shape=[
            jax.ShapeDtypeStruct((2, NN, 1024), F32),
            jax.ShapeDtypeStruct((NN, 1), F32),
        ],
    )(deg_col, x, wf1, f1b)


def _sweep_pipeline(nb, gth, gth_make, sct_make, sct_start):
    for b in range(4):
        gth(b, b)

    @pl.loop(0, nb)
    def _(b):
        sl = lax.rem(b, 6)
        gth_make(b, sl).wait()
        sct_start(b, sl)

        @pl.when(b + 4 < nb)
        def _():
            sl2 = lax.rem(b + 4, 6)

            @pl.when(b >= 2)
            def _():
                sct_make(b - 2, sl2).wait()

            gth(b + 4, sl2)

    for t in range(6):
        b = nb - 6 + t
        sct_make(b, b % 6).wait()


def _zero_acc(acc, zbuf, s, zsem):
    @pl.loop(0, 80)
    def _(z):
        pltpu.async_copy(zbuf, acc.at[pl.ds(s * 640 + z * 8, 8)], zsem)

    @pl.loop(0, 80)
    def _(z):
        pltpu.make_async_copy(zbuf, acc.at[pl.ds(s * 640 + z * 8, 8)],
                              zsem).wait()


def _fill_zbuf(zbuf):
    @pl.loop(0, 8)
    def _(i):
        @pl.loop(0, 8)
        def _(j):
            zbuf[i, pl.ds(j * 16, 16)] = jnp.zeros((16,), F32)


def _stash_body(tbl, src_e, stash, sidx, rows, gsem, wsem, ncols):
    c = lax.axis_index("c")
    s = lax.axis_index("s")
    w = c * 16 + s
    ebase = w * EPTW
    pltpu.sync_copy(src_e.at[pl.ds(ebase, EPTW)], sidx)

    def gth(b, sl):
        return pltpu.async_copy(tbl.at[sidx.at[pl.ds(b * KA, KA)]],
                                rows.at[sl], gsem.at[sl])

    def gth_make(b, sl):
        return pltpu.make_async_copy(tbl.at[sidx.at[pl.ds(b * KA, KA)]],
                                     rows.at[sl], gsem.at[sl])

    def wrt_make(b, sl):
        return pltpu.make_async_copy(
            rows.at[sl], stash.at[pl.ds(ebase + b * KA, KA), :], wsem.at[sl])

    gth(0, 0)
    gth(1, 1)

    @pl.loop(0, NBA)
    def _(b):
        sl = lax.rem(b, 4)
        gth_make(b, sl).wait()

        @pl.when(b + 2 < NBA)
        def _():
            sl2 = lax.rem(b + 2, 4)

            @pl.when(b >= 2)
            def _():
                wrt_make(b - 2, sl2).wait()

            gth(b + 2, sl2)

        pltpu.async_copy(rows.at[sl], stash.at[pl.ds(ebase + b * KA, KA), :],
                         wsem.at[sl])

    for t in range(4):
        b = NBA - 4 + t
        wrt_make(b, b % 4).wait()


@functools.partial(
    pl.kernel,
    out_type=jax.ShapeDtypeStruct((EPT, 1024), F32),
    mesh=_MESH,
    scratch_types=[
        pltpu.VMEM((EPTW,), I32),
        pltpu.VMEM((4, KA, 1024), F32),
        pltpu.SemaphoreType.DMA((4,)),
        pltpu.SemaphoreType.DMA((4,)),
    ],
)
def _stash1_kernel(rh1, src_e, stash, sidx, rows, gsem, wsem):
    _stash_body(rh1.at[1], src_e, stash, sidx, rows, gsem, wsem, 1024)


@functools.partial(
    pl.kernel,
    out_type=jax.ShapeDtypeStruct((EPT, 512), F32),
    mesh=_MESH,
    scratch_types=[
        pltpu.VMEM((EPTW,), I32),
        pltpu.VMEM((4, KA, 512), F32),
        pltpu.SemaphoreType.DMA((4,)),
        pltpu.SemaphoreType.DMA((4,)),
    ],
)
def _stash2_kernel(h2p, src_e, stash, sidx, rows, gsem, wsem):
    _stash_body(h2p, src_e, stash, sidx, rows, gsem, wsem, 512)


def _sweep_body(stash, dst_e, out, acc, didx, rows, zbuf, gsem, ssem, zsem,
                chunks_per_core):
    c = lax.axis_index("c")
    s = lax.axis_index("s")
    _fill_zbuf(zbuf)
    ebase = s * EPTS
    pltpu.sync_copy(dst_e.at[pl.ds(ebase, EPTS)], didx)

    for k in range(chunks_per_core):
        chunk = c * chunks_per_core + k
        col0 = chunk * 128
        _zero_acc(acc, zbuf, s, zsem)
        plsc.subcore_barrier()

        def gth(b, sl):
            return pltpu.async_copy(
                stash.at[pl.ds(ebase + b * K, K), pl.ds(col0, 128)],
                rows.at[sl], gsem.at[sl])

        def gth_make(b, sl):
            return pltpu.make_async_copy(
                stash.at[pl.ds(ebase + b * K, K), pl.ds(col0, 128)],
                rows.at[sl], gsem.at[sl])

        def sct_make(b, sl):
            return pltpu.make_async_copy(rows.at[sl],
                                         acc.at[didx.at[pl.ds(b * K, K)]],
                                         ssem.at[sl])

        def sct_start(b, sl):
            pltpu.async_copy(rows.at[sl], acc.at[didx.at[pl.ds(b * K, K)]],
                             ssem.at[sl], add=True)

        _sweep_pipeline(NBS, gth, gth_make, sct_make, sct_start)
        plsc.subcore_barrier()
        pltpu.sync_copy(acc.at[pl.ds(s * 640, 640)],
                        out.at[pl.ds(s * 640, 640), pl.ds(col0, 128)])
        plsc.subcore_barrier()


@functools.partial(
    pl.kernel,
    out_type=jax.ShapeDtypeStruct((NP, 1024), F32),
    mesh=_MESH,
    scratch_types=[
        pltpu.VMEM_SHARED((NP, 128), F32),
        pltpu.VMEM((EPTS,), I32),
        pltpu.VMEM((6, K, 128), F32),
        pltpu.VMEM((8, 128), F32),
        pltpu.SemaphoreType.DMA((6,)),
        pltpu.SemaphoreType.DMA((6,)),
        pltpu.SemaphoreType.DMA,
    ],
)
def _sweep1_kernel(stash, dst_e, out1, acc, didx, rows, zbuf, gsem, ssem, zsem):
    _sweep_body(stash, dst_e, out1, acc, didx, rows, zbuf, gsem, ssem, zsem, 4)


@functools.partial(
    pl.kernel,
    out_type=jax.ShapeDtypeStruct((NP, 512), F32),
    mesh=_MESH,
    scratch_types=[
        pltpu.VMEM_SHARED((NP, 128), F32),
        pltpu.VMEM((EPTS,), I32),
        pltpu.VMEM((6, K, 128), F32),
        pltpu.VMEM((8, 128), F32),
        pltpu.SemaphoreType.DMA((6,)),
        pltpu.SemaphoreType.DMA((6,)),
        pltpu.SemaphoreType.DMA,
    ],
)
def _sweep2_kernel(stash, dst_e, out2, acc, didx, rows, zbuf, gsem, ssem, zsem):
    _sweep_body(stash, dst_e, out2, acc, didx, rows, zbuf, gsem, ssem, zsem, 2)


def _gemm2_body(rh1_ref, out1_ref, dinv_ref, b1_ref, w_ref, f2b_ref,
                r2_ref, h2p_ref):
    h = jnp.maximum(
        rh1_ref[0] + dinv_ref[...] * out1_ref[...] + b1_ref[...], 0.0)
    acc = jnp.dot(h, w_ref[...], preferred_element_type=F32)
    r2_ref[...] = acc[:, :400] + f2b_ref[...]
    h2p_ref[...] = jnp.concatenate(
        [acc[:, 400:] * dinv_ref[...], jnp.zeros((h.shape[0], 112), F32)],
        axis=1)


def _gemm2(rh1, out1, dinv, b1, wf2, f2b):
    bm = 1000
    return pl.pallas_call(
        _gemm2_body,
        grid=(NN // bm,),
        in_specs=[
            pl.BlockSpec((1, bm, 1024), lambda i: (0, i, 0)),
            pl.BlockSpec((bm, 1024), lambda i: (i, 0)),
            pl.BlockSpec((bm, 1), lambda i: (i, 0)),
            pl.BlockSpec((1, 1024), lambda i: (0, 0)),
            pl.BlockSpec((1024, 800), lambda i: (0, 0)),
            pl.BlockSpec((1, 400), lambda i: (0, 0)),
        ],
        out_specs=[
            pl.BlockSpec((bm, 400), lambda i: (i, 0)),
            pl.BlockSpec((bm, 512), lambda i: (i, 0)),
        ],
        out_shape=[
            jax.ShapeDtypeStruct((NN, 400), F32),
            jax.ShapeDtypeStruct((NN, 512), F32),
        ],
    )(rh1, out1, dinv, b1, wf2, f2b)


def _comb_body(r2_ref, out2_ref, dinv_ref, b2_ref, hfin_ref):
    o = out2_ref[:, :400]
    h = jnp.maximum(r2_ref[...] + dinv_ref[...] * o + b2_ref[...], 0.0)
    hfin_ref[...] = jnp.concatenate(
        [h, jnp.zeros((h.shape[0], 112), F32)], axis=1)


def _comb(r2, out2, dinv, b2):
    bm = 2000
    return pl.pallas_call(
        _comb_body,
        grid=(NN // bm,),
        in_specs=[
            pl.BlockSpec((bm, 400), lambda i: (i, 0)),
            pl.BlockSpec((bm, 512), lambda i: (i, 0)),
            pl.BlockSpec((bm, 1), lambda i: (i, 0)),
            pl.BlockSpec((1, 400), lambda i: (0, 0)),
        ],
        out_specs=pl.BlockSpec((bm, 512), lambda i: (i, 0)),
        out_shape=jax.ShapeDtypeStruct((NN, 512), F32),
    )(r2, out2, dinv, b2)


@functools.partial(
    pl.kernel,
    out_type=jax.ShapeDtypeStruct((4096, 1024), F32),
    mesh=_MESH,
    scratch_types=[
        pltpu.VMEM((128,), I32),
        pltpu.VMEM((128,), I32),
        pltpu.VMEM((128, 512), F32),
        pltpu.SemaphoreType.DMA((1,)),
    ],
)
def _pairs_kernel(hfin, lidx, ridx, feat, li, ri, buf, sems):
    c = lax.axis_index("c")
    s = lax.axis_index("s")
    w = c * 16 + s
    p0 = w * 128
    pltpu.sync_copy(lidx.at[pl.ds(p0, 128)], li)
    pltpu.sync_copy(ridx.at[pl.ds(p0, 128)], ri)
    pltpu.async_copy(hfin.at[li], buf, sems.at[0])
    pltpu.make_async_copy(hfin.at[li], buf, sems.at[0]).wait()
    pltpu.sync_copy(buf, feat.at[pl.ds(p0, 128), pl.ds(0, 512)])
    pltpu.async_copy(hfin.at[ri], buf, sems.at[0])
    pltpu.make_async_copy(hfin.at[ri], buf, sems.at[0]).wait()
    pltpu.sync_copy(buf, feat.at[pl.ds(p0, 128), pl.ds(512, 512)])


def _mlp_body(feat_ref, w1_ref, b1_ref, w2_ref, b2_ref, w3_ref, b3_ref, z_ref):
    z1 = (jnp.dot(feat_ref[:, :400], w1_ref[:400], preferred_element_type=F32)
          + jnp.dot(feat_ref[:, 512:912], w1_ref[400:],
                    preferred_element_type=F32))
    z1 = jnp.maximum(z1 + b1_ref[...], 0.0)
    z2 = jnp.maximum(
        jnp.dot(z1, w2_ref[...], preferred_element_type=F32) + b2_ref[...], 0.0)
    z3 = jnp.dot(z2, w3_ref[...], preferred_element_type=F32) + b3_ref[...]
    z_ref[...] = jax.nn.sigmoid(z3)


def _mlp(feat, w1, b1, w2, b2, w3, b3):
    return pl.pallas_call(
        _mlp_body,
        out_shape=jax.ShapeDtypeStruct((4096, 1), F32),
    )(feat, w1, b1, w2, b2, w3, b3)


def kernel(x, a, sample_train, conv1_W, conv1_b, conv2_W, conv2_b,
           f1_W, f1_b, f2_W, f2_b, fc1_W, fc1_b, fc2_W, fc2_b, fc3_W, fc3_b):
    loop = jnp.arange(NN, dtype=I32)
    pad = EPT - NE - NN
    src_e = jnp.concatenate([a[0], loop, jnp.zeros((pad,), I32)])
    dst_e = jnp.concatenate([a[1], loop, jnp.full((pad,), TRASH, I32)])

    deg0, deg1 = _deg_kernel(dst_e)
    deg_col = (deg0 + deg1)[:, None]

    wf1 = jnp.concatenate([f1_W, conv1_W], axis=1)
    rh1, dinv = _gemm1(deg_col, x, wf1, f1_b.reshape(1, -1))

    stash1 = _stash1_kernel(rh1, src_e)
    out1 = _sweep1_kernel(stash1, dst_e)

    wf2 = jnp.concatenate([f2_W, conv2_W], axis=1)
    r2, h2p = _gemm2(rh1, out1, dinv, conv1_b.reshape(1, -1), wf2,
                     f2_b.reshape(1, -1))

    stash2 = _stash2_kernel(h2p, src_e)
    out2 = _sweep2_kernel(stash2, dst_e)
    hfin = _comb(r2, out2, dinv, conv2_b.reshape(1, -1))

    lidx = sample_train[:, 0]
    ridx = sample_train[:, 1] + 5000
    feat = _pairs_kernel(hfin, lidx, ridx)

    z = _mlp(feat, fc1_W, fc1_b.reshape(1, -1), fc2_W, fc2_b.reshape(1, -1),
             fc3_W, fc3_b.reshape(1, -1))
    return z
